# baseline (speedup 1.0000x reference)
import jax
import jax.numpy as jnp
from jax.experimental import pallas as pl
from jax.experimental.pallas import tpu as pltpu

_TEMP = 0.1
_N = 2048
_NBINS = 16
_K = 4
_H = _N // _K


def _rank_loss_kernel(bi_ref, z_ref, out_ref,
                      rcol_ref, rrow_ref, vrow_ref, trow_ref, tcol_ref,
                      const_ref):
    s = pl.program_id(0)

    zb = z_ref[...]
    rb = jnp.sqrt(jnp.sum(zb * zb, axis=1, keepdims=True)) * (0.5 / _TEMP)
    rcol_ref[pl.ds(s * _H, _H), :] = rb
    rrow_ref[:, pl.ds(s * _H, _H)] = jnp.transpose(rb, (1, 0))

    @pl.when(s == 0)
    def _():
        m = bi_ref[...].astype(jnp.float32)
        v = jnp.zeros(m.shape, dtype=jnp.float32)
        for _ in range(13):
            q = jnp.round(m * (1.0 / 3.0))
            div = (m > 0.0) & (q * 3.0 == m)
            v = v + div.astype(jnp.float32)
            m = jnp.where(div, q, m)
        vrow_ref[...] = v

        bins = jax.lax.broadcasted_iota(jnp.int32, (_NBINS, 1), 0).astype(jnp.float32)
        n_b = jnp.sum((bins == v).astype(jnp.float32), axis=1, keepdims=True)
        w_b = jnp.sum(jnp.maximum(bins - v, 0.0), axis=1, keepdims=True)
        const_ref[0, 0] = jnp.sum(n_b * w_b)
        const_ref[0, 1] = float(_N * _N) - jnp.sum(n_b * n_b)
        tcol_ref[...] = jnp.zeros_like(tcol_ref)

    ones_row = jnp.ones((1, _H), dtype=jnp.float32)
    ones_col = jnp.ones((_H, 1), dtype=jnp.float32)
    rr = rrow_ref[:, pl.ds(s * _H, _H)]
    for p in range(_K):

        @pl.when(p <= s)
        def _(p=p):
            tb = jnp.tanh(rr - rcol_ref[pl.ds(p * _H, _H), :])
            csum = jax.lax.dot_general(
                ones_row, tb, (((1,), (0,)), ((), ())),
                preferred_element_type=jnp.float32)
            if p == 0:
                trow_ref[:, pl.ds(s * _H, _H)] = csum
            else:
                trow_ref[:, pl.ds(s * _H, _H)] += csum

            if p < _K - 1:

                @pl.when(p < s)
                def _(p=p, tb=tb):
                    tcol_ref[pl.ds(p * _H, _H), :] -= jax.lax.dot_general(
                        tb, ones_col, (((1,), (0,)), ((), ())),
                        preferred_element_type=jnp.float32)

    @pl.when(s == _K - 1)
    def _():
        t_all = trow_ref[...] + jnp.transpose(tcol_ref[...], (1, 0))
        num = jnp.sum(vrow_ref[...] * t_all) + const_ref[0, 0]
        out_ref[0, 0] = num / jnp.maximum(const_ref[0, 1], 1.0)


def kernel(z_hyp, batch_indices):
    loss = pl.pallas_call(
        _rank_loss_kernel,
        grid=(_K,),
        in_specs=[
            pl.BlockSpec((1, _N), lambda i: (0, 0)),
            pl.BlockSpec((_H, 128), lambda i: (i, 0)),
        ],
        out_specs=pl.BlockSpec(block_shape=(1, 1), index_map=lambda i: (0, 0),
                               memory_space=pltpu.SMEM),
        out_shape=jax.ShapeDtypeStruct((1, 1), jnp.float32),
        scratch_shapes=[
            pltpu.VMEM((_N, 1), jnp.float32),
            pltpu.VMEM((1, _N), jnp.float32),
            pltpu.VMEM((1, _N), jnp.float32),
            pltpu.VMEM((1, _N), jnp.float32),
            pltpu.VMEM((_N, 1), jnp.float32),
            pltpu.SMEM((1, 2), jnp.float32),
        ],
    )(batch_indices.reshape(1, _N), z_hyp)
    return loss[0, 0]


# speedup vs baseline: 2.0905x; 2.0905x over previous
import jax
import jax.numpy as jnp
from jax.experimental import pallas as pl
from jax.experimental.pallas import tpu as pltpu

_TEMP = 0.1
_N = 2048
_NBINS = 16
_K = 4
_H = _N // _K


def _rank_loss_kernel(z_ref, bi_ref, out_ref):
    z = z_ref[...]
    rcol = jnp.sqrt(jnp.sum(z * z, axis=1, keepdims=True)) * (0.5 / _TEMP)
    rrow = jnp.transpose(rcol, (1, 0))

    m = bi_ref[...].reshape(1, _N).astype(jnp.float32)
    v = jnp.zeros(m.shape, dtype=jnp.float32)
    for _ in range(13):
        q = jnp.round(m * (1.0 / 3.0))
        div = (m > 0.0) & (q * 3.0 == m)
        v = v + div.astype(jnp.float32)
        m = jnp.where(div, q, m)

    bins = jax.lax.broadcasted_iota(jnp.int32, (_NBINS, 1), 0).astype(jnp.float32)
    n_b = jnp.sum((bins == v).astype(jnp.float32), axis=1, keepdims=True)
    w_b = jnp.sum(jnp.maximum(bins - v, 0.0), axis=1, keepdims=True)
    c_const = jnp.sum(n_b * w_b)
    denom = float(_N * _N) - jnp.sum(n_b * n_b)

    ones_row = jnp.ones((1, _H), dtype=jnp.float32)
    ones_col = jnp.ones((_H, 1), dtype=jnp.float32)
    trow = [jnp.zeros((1, _H), dtype=jnp.float32) for _ in range(_K)]
    tcol = [jnp.zeros((_H, 1), dtype=jnp.float32) for _ in range(_K)]
    for q in range(_K):
        rr = rrow[:, q * _H:(q + 1) * _H]
        for p in range(q + 1):
            tb = jnp.tanh(rr - rcol[p * _H:(p + 1) * _H, :])
            trow[q] = trow[q] + jax.lax.dot_general(
                ones_row, tb, (((1,), (0,)), ((), ())),
                preferred_element_type=jnp.float32)
            if p < q:
                tcol[p] = tcol[p] - jax.lax.dot_general(
                    tb, ones_col, (((1,), (0,)), ((), ())),
                    preferred_element_type=jnp.float32)

    num = c_const
    for p in range(_K):
        t_p = trow[p] + jnp.transpose(tcol[p], (1, 0))
        num = num + jnp.sum(v[:, p * _H:(p + 1) * _H] * t_p)
    out_ref[0, 0] = num / jnp.maximum(denom, 1.0)


def kernel(z_hyp, batch_indices):
    loss = pl.pallas_call(
        _rank_loss_kernel,
        in_specs=[
            pl.BlockSpec((_N, 128), lambda: (0, 0)),
            pl.BlockSpec((_N,), lambda: (0,)),
        ],
        out_specs=pl.BlockSpec(block_shape=(1, 1), index_map=lambda: (0, 0),
                               memory_space=pltpu.SMEM),
        out_shape=jax.ShapeDtypeStruct((1, 1), jnp.float32),
    )(z_hyp, batch_indices)
    return loss[0, 0]
